# baseline (speedup 1.0000x reference)
import functools

import jax
import jax.numpy as jnp
from jax import lax
from jax.experimental import pallas as pl
from jax.experimental.pallas import tpu as pltpu
from jax.experimental.pallas import tpu_sc as plsc

N = 10000
E = 320000
DX = 128
DE = 16

NC = 2
NS = 16

HALF = E // NC
CH = 16000
NPASS = HALF // CH
GROUPS = CH // 16
N_PAD = 10240
ZGROUPS = N_PAD // 16


def _sc_segment_sum_t(idx1d, attr_t):
    mesh = plsc.VectorSubcoreMesh(core_axis_name="c", subcore_axis_name="s")

    @functools.partial(
        pl.kernel,
        mesh=mesh,
        out_type=jax.ShapeDtypeStruct((NC, DE, N_PAD), jnp.float32),
        scratch_types=[
            pltpu.VMEM((CH // 128, 128), jnp.int32),
            pltpu.VMEM((CH // 128, 128), jnp.int32),
            pltpu.VMEM((CH // 128, 128), jnp.float32),
            pltpu.VMEM((CH // 128, 128), jnp.float32),
            pltpu.VMEM((N_PAD,), jnp.float32),
            pltpu.SemaphoreType.DMA,
            pltpu.SemaphoreType.DMA,
        ],
        compiler_params=pltpu.CompilerParams(use_tc_tiling_on_sc=False,
                                             needs_layout_passes=False),
    )
    def sc_kernel(idx_hbm, attr_hbm, out_hbm, idx0, idx1, val0, val1, plane,
                  sem0, sem1):
        cid = lax.axis_index("c")
        f = lax.axis_index("s")
        fa = f // 8
        fc = f % 8
        tbase = cid * (HALF // 128)

        zvec = jnp.zeros((16,), jnp.float32)

        def zbody(i, carry):
            plane[pl.ds(16 * i, 16)] = zvec
            return carry

        lax.fori_loop(0, ZGROUPS, zbody, 0)

        idxb = (idx0, idx1)
        valb = (val0, val1)
        sems = (sem0, sem1)

        TPP = CH // 128

        def start(p):
            b = p % 2
            ci = pltpu.async_copy(
                idx_hbm.at[pl.ds(tbase + p * TPP, TPP), 0, :], idxb[b],
                sems[b])
            cv = pltpu.async_copy(
                attr_hbm.at[fa, pl.ds(tbase + p * TPP, TPP), fc, :], valb[b],
                sems[b])
            return ci, cv

        cps = {0: start(0)}
        for p in range(NPASS):
            b = p % 2
            if p + 1 < NPASS:
                cps[(p + 1) % 2] = start(p + 1)
            ci, cv = cps[b]
            ci.wait()
            cv.wait()
            ib, vb = idxb[b], valb[b]

            @plsc.parallel_loop(0, CH // 128, unroll=4)
            def inner(t, ib=ib, vb=vb):
                for g in range(8):
                    idxv = ib[t, pl.ds(16 * g, 16)]
                    vals = vb[t, pl.ds(16 * g, 16)]
                    plsc.addupdate_scatter(plane, [idxv], vals)

        pltpu.sync_copy(plane, out_hbm.at[cid, f])

    return sc_kernel(idx1d, attr_t)


def _tc_linear_t(x, partials, Wx, bxc, We, bec):
    L = 2048
    grid = (pl.cdiv(N, L),)

    def body(x_ref, p_ref, wx_ref, bx_ref, we_ref, be_ref, o_ref):
        hx = lax.dot_general(wx_ref[...], x_ref[...],
                             (((1,), (1,)), ((), ())),
                             preferred_element_type=jnp.float32)
        hx = hx + bx_ref[...]
        aggt = p_ref[0] + p_ref[1]
        he = lax.dot_general(we_ref[...], aggt,
                             (((1,), (0,)), ((), ())),
                             preferred_element_type=jnp.float32)
        he = he + be_ref[...]
        o_ref[:DX, :] = jnp.maximum(hx, 0.0)
        o_ref[DX:, :] = jnp.maximum(he, 0.0)

    return pl.pallas_call(
        body,
        grid=grid,
        in_specs=[
            pl.BlockSpec((L, DX), lambda i: (i, 0)),
            pl.BlockSpec((NC, DE, L), lambda i: (0, 0, i)),
            pl.BlockSpec((DX, DX), lambda i: (0, 0)),
            pl.BlockSpec((DX, 1), lambda i: (0, 0)),
            pl.BlockSpec((DE, DE), lambda i: (0, 0)),
            pl.BlockSpec((DE, 1), lambda i: (0, 0)),
        ],
        out_specs=pl.BlockSpec((DX + DE, L), lambda i: (0, i)),
        out_shape=jax.ShapeDtypeStruct((DX + DE, N), jnp.float32),
    )(x, partials, Wx, bxc, We, bec)


def kernel(x, edge_index, edge_attr, Wx, bx, We, be):
    idx3 = edge_index.astype(jnp.int32).T.reshape(E // 128, 128, 2)
    idx3 = idx3.transpose(0, 2, 1)
    attr4 = edge_attr.T.reshape(2, 8, E // 128, 128).transpose(0, 2, 1, 3)
    partials = _sc_segment_sum_t(idx3, attr4)
    out_t = _tc_linear_t(x, partials, Wx, bx.reshape(DX, 1), We,
                         be.reshape(DE, 1))
    return out_t.T


# speedup vs baseline: 1.0022x; 1.0022x over previous
import functools

import jax
import jax.numpy as jnp
from jax import lax
from jax.experimental import pallas as pl
from jax.experimental.pallas import tpu as pltpu
from jax.experimental.pallas import tpu_sc as plsc

N = 10000
E = 320000
DX = 128
DE = 16

NC = 2
NS = 16

HALF = E // NC
CH = 16000
NPASS = HALF // CH
GROUPS = CH // 16
N_PAD = 10240
ZGROUPS = N_PAD // 16


def _sc_segment_sum_t(idx1d, attr_t):
    mesh = plsc.VectorSubcoreMesh(core_axis_name="c", subcore_axis_name="s")

    @functools.partial(
        pl.kernel,
        mesh=mesh,
        out_type=jax.ShapeDtypeStruct((NC, DE, N_PAD), jnp.float32),
        scratch_types=[
            pltpu.VMEM((CH // 128, 128), jnp.int32),
            pltpu.VMEM((CH // 128, 128), jnp.int32),
            pltpu.VMEM((CH // 128, 128), jnp.float32),
            pltpu.VMEM((CH // 128, 128), jnp.float32),
            pltpu.VMEM((N_PAD,), jnp.float32),
            pltpu.VMEM((N_PAD,), jnp.float32),
            pltpu.SemaphoreType.DMA,
            pltpu.SemaphoreType.DMA,
        ],
        compiler_params=pltpu.CompilerParams(use_tc_tiling_on_sc=False,
                                             needs_layout_passes=False),
    )
    def sc_kernel(idx_hbm, attr_hbm, out_hbm, idx0, idx1, val0, val1, plane,
                  planeb, sem0, sem1):
        cid = lax.axis_index("c")
        f = lax.axis_index("s")
        fa = f // 8
        fc = f % 8
        tbase = cid * (HALF // 128)

        zvec = jnp.zeros((16,), jnp.float32)

        def zbody(i, carry):
            plane[pl.ds(16 * i, 16)] = zvec
            planeb[pl.ds(16 * i, 16)] = zvec
            return carry

        lax.fori_loop(0, ZGROUPS, zbody, 0)

        idxb = (idx0, idx1)
        valb = (val0, val1)
        sems = (sem0, sem1)

        TPP = CH // 128

        def start(p):
            b = p % 2
            ci = pltpu.async_copy(
                idx_hbm.at[pl.ds(tbase + p * TPP, TPP), 0, :], idxb[b],
                sems[b])
            cv = pltpu.async_copy(
                attr_hbm.at[fa, pl.ds(tbase + p * TPP, TPP), fc, :], valb[b],
                sems[b])
            return ci, cv

        cps = {0: start(0)}
        for p in range(NPASS):
            b = p % 2
            if p + 1 < NPASS:
                cps[(p + 1) % 2] = start(p + 1)
            ci, cv = cps[b]
            ci.wait()
            cv.wait()
            ib, vb = idxb[b], valb[b]

            @plsc.parallel_loop(0, CH // 128, unroll=2)
            def inner(t, ib=ib, vb=vb):
                for g in range(8):
                    idxv = ib[t, pl.ds(16 * g, 16)]
                    vals = vb[t, pl.ds(16 * g, 16)]
                    plsc.addupdate_scatter((plane, planeb)[g % 2],
                                           [idxv], vals)

        @plsc.parallel_loop(0, ZGROUPS, unroll=8)
        def merge(i):
            sl = pl.ds(16 * i, 16)
            plane[sl] = plane[sl] + planeb[sl]

        pltpu.sync_copy(plane, out_hbm.at[cid, f])

    return sc_kernel(idx1d, attr_t)


def _tc_linear_t(x, partials, Wx, bxc, We, bec):
    L = 2048
    grid = (pl.cdiv(N, L),)

    def body(x_ref, p_ref, wx_ref, bx_ref, we_ref, be_ref, o_ref):
        hx = lax.dot_general(wx_ref[...], x_ref[...],
                             (((1,), (1,)), ((), ())),
                             preferred_element_type=jnp.float32)
        hx = hx + bx_ref[...]
        aggt = p_ref[0] + p_ref[1]
        he = lax.dot_general(we_ref[...], aggt,
                             (((1,), (0,)), ((), ())),
                             preferred_element_type=jnp.float32)
        he = he + be_ref[...]
        o_ref[:DX, :] = jnp.maximum(hx, 0.0)
        o_ref[DX:, :] = jnp.maximum(he, 0.0)

    return pl.pallas_call(
        body,
        grid=grid,
        in_specs=[
            pl.BlockSpec((L, DX), lambda i: (i, 0)),
            pl.BlockSpec((NC, DE, L), lambda i: (0, 0, i)),
            pl.BlockSpec((DX, DX), lambda i: (0, 0)),
            pl.BlockSpec((DX, 1), lambda i: (0, 0)),
            pl.BlockSpec((DE, DE), lambda i: (0, 0)),
            pl.BlockSpec((DE, 1), lambda i: (0, 0)),
        ],
        out_specs=pl.BlockSpec((DX + DE, L), lambda i: (0, i)),
        out_shape=jax.ShapeDtypeStruct((DX + DE, N), jnp.float32),
    )(x, partials, Wx, bxc, We, bec)


def kernel(x, edge_index, edge_attr, Wx, bx, We, be):
    idx3 = edge_index.astype(jnp.int32).T.reshape(E // 128, 128, 2)
    idx3 = idx3.transpose(0, 2, 1)
    attr4 = edge_attr.T.reshape(2, 8, E // 128, 128).transpose(0, 2, 1, 3)
    partials = _sc_segment_sum_t(idx3, attr4)
    out_t = _tc_linear_t(x, partials, Wx, bx.reshape(DX, 1), We,
                         be.reshape(DE, 1))
    return out_t.T


# speedup vs baseline: 1.0570x; 1.0548x over previous
import functools

import jax
import jax.numpy as jnp
from jax import lax
from jax.experimental import pallas as pl
from jax.experimental.pallas import tpu as pltpu
from jax.experimental.pallas import tpu_sc as plsc

N = 10000
E = 320000
DX = 128
DE = 16

NC = 2
NS = 16

HALF = E // NC
QUART = E // 4
CH = 16000
NPASS = QUART // CH
N_PAD = 10240
ZGROUPS = N_PAD // 16


def _sc_segment_sum_t(idx1d, attr_t):
    mesh = plsc.VectorSubcoreMesh(core_axis_name="c", subcore_axis_name="s")

    @functools.partial(
        pl.kernel,
        mesh=mesh,
        out_type=jax.ShapeDtypeStruct((NC, 2, DE, N_PAD), jnp.float32),
        scratch_types=[
            pltpu.VMEM((CH // 128, 128), jnp.int32),
            pltpu.VMEM((CH // 128, 128), jnp.int32),
            pltpu.VMEM((CH // 128, 128), jnp.float32),
            pltpu.VMEM((CH // 128, 128), jnp.float32),
            pltpu.VMEM((CH // 128, 128), jnp.float32),
            pltpu.VMEM((CH // 128, 128), jnp.float32),
            pltpu.VMEM((N_PAD,), jnp.float32),
            pltpu.VMEM((N_PAD,), jnp.float32),
            pltpu.SemaphoreType.DMA,
            pltpu.SemaphoreType.DMA,
        ],
        compiler_params=pltpu.CompilerParams(use_tc_tiling_on_sc=False,
                                             needs_layout_passes=False),
    )
    def sc_kernel(idx_hbm, attr_hbm, out_hbm, idx0, idx1, vala0, vala1,
                  valb0, valb1, plane_a, plane_b, sem0, sem1):
        cid = lax.axis_index("c")
        s = lax.axis_index("s")
        half = s // 8
        fc = s % 8
        tbase = (cid * 2 + half) * (QUART // 128)

        zvec = jnp.zeros((16,), jnp.float32)

        def zbody(i, carry):
            plane_a[pl.ds(16 * i, 16)] = zvec
            plane_b[pl.ds(16 * i, 16)] = zvec
            return carry

        lax.fori_loop(0, ZGROUPS, zbody, 0)

        idxb = (idx0, idx1)
        valab = (vala0, vala1)
        valbb = (valb0, valb1)
        sems = (sem0, sem1)

        TPP = CH // 128

        def start(p):
            b = p % 2
            ci = pltpu.async_copy(
                idx_hbm.at[pl.ds(tbase + p * TPP, TPP), 0, :], idxb[b],
                sems[b])
            cva = pltpu.async_copy(
                attr_hbm.at[0, pl.ds(tbase + p * TPP, TPP), fc, :], valab[b],
                sems[b])
            cvb = pltpu.async_copy(
                attr_hbm.at[1, pl.ds(tbase + p * TPP, TPP), fc, :], valbb[b],
                sems[b])
            return ci, cva, cvb

        cps = {0: start(0)}
        for p in range(NPASS):
            b = p % 2
            if p + 1 < NPASS:
                cps[(p + 1) % 2] = start(p + 1)
            ci, cva, cvb = cps[b]
            ci.wait()
            cva.wait()
            cvb.wait()
            ib, va, vb = idxb[b], valab[b], valbb[b]

            @plsc.parallel_loop(0, CH // 128, unroll=2)
            def inner(t, ib=ib, va=va, vb=vb):
                for g in range(8):
                    sl = pl.ds(16 * g, 16)
                    idxv = ib[t, sl]
                    plsc.addupdate_scatter(plane_a, [idxv], va[t, sl])
                    plsc.addupdate_scatter(plane_b, [idxv], vb[t, sl])

        pltpu.sync_copy(plane_a, out_hbm.at[cid, half, fc])
        pltpu.sync_copy(plane_b, out_hbm.at[cid, half, fc + 8])

    return sc_kernel(idx1d, attr_t)


def _tc_linear_t(x, partials, Wx, bxc, We, bec):
    L = 2048
    grid = (pl.cdiv(N, L),)

    def body(x_ref, p_ref, wx_ref, bx_ref, we_ref, be_ref, o_ref):
        hx = lax.dot_general(wx_ref[...], x_ref[...],
                             (((1,), (1,)), ((), ())),
                             preferred_element_type=jnp.float32)
        hx = hx + bx_ref[...]
        aggt = ((p_ref[0, 0] + p_ref[0, 1]) + (p_ref[1, 0] + p_ref[1, 1]))
        he = lax.dot_general(we_ref[...], aggt,
                             (((1,), (0,)), ((), ())),
                             preferred_element_type=jnp.float32)
        he = he + be_ref[...]
        o_ref[:DX, :] = jnp.maximum(hx, 0.0)
        o_ref[DX:, :] = jnp.maximum(he, 0.0)

    return pl.pallas_call(
        body,
        grid=grid,
        in_specs=[
            pl.BlockSpec((L, DX), lambda i: (i, 0)),
            pl.BlockSpec((NC, 2, DE, L), lambda i: (0, 0, 0, i)),
            pl.BlockSpec((DX, DX), lambda i: (0, 0)),
            pl.BlockSpec((DX, 1), lambda i: (0, 0)),
            pl.BlockSpec((DE, DE), lambda i: (0, 0)),
            pl.BlockSpec((DE, 1), lambda i: (0, 0)),
        ],
        out_specs=pl.BlockSpec((DX + DE, L), lambda i: (0, i)),
        out_shape=jax.ShapeDtypeStruct((DX + DE, N), jnp.float32),
    )(x, partials, Wx, bxc, We, bec)


def kernel(x, edge_index, edge_attr, Wx, bx, We, be):
    idx3 = edge_index.astype(jnp.int32).T.reshape(E // 128, 128, 2)
    idx3 = idx3.transpose(0, 2, 1)
    attr4 = edge_attr.T.reshape(2, 8, E // 128, 128).transpose(0, 2, 1, 3)
    partials = _sc_segment_sum_t(idx3, attr4)
    out_t = _tc_linear_t(x, partials, Wx, bx.reshape(DX, 1), We,
                         be.reshape(DE, 1))
    return out_t.T


# speedup vs baseline: 1.1067x; 1.0469x over previous
import functools

import jax
import jax.numpy as jnp
from jax import lax
from jax.experimental import pallas as pl
from jax.experimental.pallas import tpu as pltpu
from jax.experimental.pallas import tpu_sc as plsc

N = 10000
E = 320000
DX = 128
DE = 16

NC = 2
NS = 16

HALF = E // NC
QUART = E // 4
CH = 16000
NPASS = QUART // CH
N_PAD = 10240
ZGROUPS = N_PAD // 16


def _sc_segment_sum_t(idx1d, attr_t):
    mesh = plsc.VectorSubcoreMesh(core_axis_name="c", subcore_axis_name="s")

    @functools.partial(
        pl.kernel,
        mesh=mesh,
        out_type=jax.ShapeDtypeStruct((NC, 2, DE, N_PAD), jnp.float32),
        scratch_types=[
            pltpu.VMEM((CH // 128, 128), jnp.int32),
            pltpu.VMEM((CH // 128, 128), jnp.int32),
            pltpu.VMEM((CH // 128, 128), jnp.float32),
            pltpu.VMEM((CH // 128, 128), jnp.float32),
            pltpu.VMEM((CH // 128, 128), jnp.float32),
            pltpu.VMEM((CH // 128, 128), jnp.float32),
            pltpu.VMEM((N_PAD,), jnp.float32),
            pltpu.VMEM((N_PAD,), jnp.float32),
            pltpu.SemaphoreType.DMA,
            pltpu.SemaphoreType.DMA,
        ],
        compiler_params=pltpu.CompilerParams(use_tc_tiling_on_sc=False,
                                             needs_layout_passes=False),
    )
    def sc_kernel(idx_hbm, attr_hbm, out_hbm, idx0, idx1, vala0, vala1,
                  valb0, valb1, plane_a, plane_b, sem0, sem1):
        cid = lax.axis_index("c")
        s = lax.axis_index("s")
        half = s // 8
        fc = s % 8
        tbase = (cid * 2 + half) * (QUART // 128)

        zvec = jnp.zeros((16,), jnp.float32)

        def zbody(i, carry):
            plane_a[pl.ds(16 * i, 16)] = zvec
            plane_b[pl.ds(16 * i, 16)] = zvec
            return carry

        lax.fori_loop(0, ZGROUPS, zbody, 0)

        idxb = (idx0, idx1)
        valab = (vala0, vala1)
        valbb = (valb0, valb1)
        sems = (sem0, sem1)

        TPP = CH // 128

        def start(p):
            b = p % 2
            ci = pltpu.async_copy(
                idx_hbm.at[pl.ds(tbase + p * TPP, TPP), 0, :], idxb[b],
                sems[b])
            cva = pltpu.async_copy(
                attr_hbm.at[0, pl.ds(tbase + p * TPP, TPP), fc, :], valab[b],
                sems[b])
            cvb = pltpu.async_copy(
                attr_hbm.at[1, pl.ds(tbase + p * TPP, TPP), fc, :], valbb[b],
                sems[b])
            return ci, cva, cvb

        cps = {0: start(0)}
        for p in range(NPASS):
            b = p % 2
            if p + 1 < NPASS:
                cps[(p + 1) % 2] = start(p + 1)
            ci, cva, cvb = cps[b]
            ci.wait()
            cva.wait()
            cvb.wait()
            ib, va, vb = idxb[b], valab[b], valbb[b]

            @plsc.parallel_loop(0, CH // 128, unroll=2)
            def inner(t, ib=ib, va=va, vb=vb):
                for g in range(8):
                    sl = pl.ds(16 * g, 16)
                    idxv = ib[t, sl]
                    plsc.addupdate_scatter(plane_a, [idxv], va[t, sl])
                    plsc.addupdate_scatter(plane_b, [idxv], vb[t, sl])

        pltpu.sync_copy(plane_a, out_hbm.at[cid, half, fc])
        pltpu.sync_copy(plane_b, out_hbm.at[cid, half, fc + 8])

    return sc_kernel(idx1d, attr_t)


def _tc_hx_t(x, Wx, bxc):
    L = 2048

    def body(x_ref, wx_ref, bx_ref, o_ref):
        hx = lax.dot_general(wx_ref[...], x_ref[...],
                             (((1,), (1,)), ((), ())),
                             preferred_element_type=jnp.float32)
        o_ref[...] = jnp.maximum(hx + bx_ref[...], 0.0)

    return pl.pallas_call(
        body,
        grid=(pl.cdiv(N, L),),
        in_specs=[
            pl.BlockSpec((L, DX), lambda i: (i, 0)),
            pl.BlockSpec((DX, DX), lambda i: (0, 0)),
            pl.BlockSpec((DX, 1), lambda i: (0, 0)),
        ],
        out_specs=pl.BlockSpec((DX, L), lambda i: (0, i)),
        out_shape=jax.ShapeDtypeStruct((DX + DE, N), jnp.float32),
    )(x, Wx, bxc)


def _tc_he_t(partials, We, bec, base):
    L = 2048

    def body(p_ref, we_ref, be_ref, base_ref, o_ref):
        del base_ref
        aggt = ((p_ref[0, 0] + p_ref[0, 1]) + (p_ref[1, 0] + p_ref[1, 1]))
        he = lax.dot_general(we_ref[...], aggt,
                             (((1,), (0,)), ((), ())),
                             preferred_element_type=jnp.float32)
        o_ref[...] = jnp.maximum(he + be_ref[...], 0.0)

    return pl.pallas_call(
        body,
        grid=(pl.cdiv(N, L),),
        in_specs=[
            pl.BlockSpec((NC, 2, DE, L), lambda i: (0, 0, 0, i)),
            pl.BlockSpec((DE, DE), lambda i: (0, 0)),
            pl.BlockSpec((DE, 1), lambda i: (0, 0)),
            pl.BlockSpec(memory_space=pltpu.MemorySpace.HBM),
        ],
        out_specs=pl.BlockSpec((DE, L), lambda i: (DX // DE, i)),
        out_shape=jax.ShapeDtypeStruct((DX + DE, N), jnp.float32),
        input_output_aliases={3: 0},
    )(partials, We, bec, base)


def kernel(x, edge_index, edge_attr, Wx, bx, We, be):
    idx3 = edge_index.astype(jnp.int32).T.reshape(E // 128, 128, 2)
    idx3 = idx3.transpose(0, 2, 1)
    attr4 = edge_attr.T.reshape(2, 8, E // 128, 128).transpose(0, 2, 1, 3)
    partials = _sc_segment_sum_t(idx3, attr4)
    base = _tc_hx_t(x, Wx, bx.reshape(DX, 1))
    out_t = _tc_he_t(partials, We, be.reshape(DE, 1), base)
    return out_t.T
